# speedup vs baseline: 1.7707x; 1.7707x over previous
import functools

import jax
import jax.numpy as jnp
from jax import lax
from jax.experimental import pallas as pl
from jax.experimental.pallas import tpu as pltpu
from jax.experimental.pallas import tpu_sc as plsc

_FEAT = 2048
_NCLS = 1000
_BATCH = 16384
_NC = 2
_NS = 16
_NW = _NC * _NS
_BPW = _BATCH // _NW
_CHUNK = 16
_NCHUNK = _BPW // _CHUNK


def _sc_gather(table, idx):
    mesh = plsc.VectorSubcoreMesh(core_axis_name="c", subcore_axis_name="s")

    @functools.partial(
        pl.kernel,
        out_type=jax.ShapeDtypeStruct((_BATCH, _FEAT), jnp.float32),
        mesh=mesh,
        scratch_types=[
            pltpu.VMEM((_BPW,), jnp.int32),
            pltpu.VMEM((_CHUNK, _FEAT), jnp.float32),
            pltpu.VMEM((_CHUNK, _FEAT), jnp.float32),
            pltpu.SemaphoreType.DMA,
            pltpu.SemaphoreType.DMA,
        ],
    )
    def k(table_hbm, idx_hbm, out_hbm, idx_v, buf0, buf1, sem0, sem1):
        wid = lax.axis_index("s") * _NC + lax.axis_index("c")
        base = wid * _BPW
        pltpu.sync_copy(idx_hbm.at[pl.ds(base, _BPW)], idx_v)

        bufs = (buf0, buf1)
        sems = (sem0, sem1)

        pltpu.async_copy(table_hbm.at[idx_v.at[pl.ds(0, _CHUNK)]], buf0, sem0)
        pltpu.async_copy(table_hbm.at[idx_v.at[pl.ds(_CHUNK, _CHUNK)]], buf1, sem1)

        @pl.loop(0, _NCHUNK, step=2)
        def _(g0):
            for b in range(2):
                g = g0 + b
                buf, sem = bufs[b], sems[b]
                pltpu.make_async_copy(
                    table_hbm.at[idx_v.at[pl.ds(0, _CHUNK)]], buf, sem
                ).wait()
                pltpu.sync_copy(
                    buf, out_hbm.at[pl.ds(base + g * _CHUNK, _CHUNK)]
                )

                @pl.when(g + 2 < _NCHUNK)
                def _():
                    pltpu.async_copy(
                        table_hbm.at[idx_v.at[pl.ds((g + 2) * _CHUNK, _CHUNK)]],
                        buf,
                        sem,
                    )

    return k(table, idx)


def kernel(label, proto):
    table = proto.T
    return _sc_gather(table, label.astype(jnp.int32))
